# baseline (device time: 392896 ns/iter reference)
import jax
import jax.numpy as jnp
from jax import lax
from jax.experimental import pallas as pl
from jax.experimental.pallas import tpu as pltpu

N_DEV = 8
M, NOUT = 4096, 2048
CH = M // N_DEV
NH = NOUT // 2
NQ = NH // 2
LAST_CREDIT = 11


def _ring_pos(i):
    return jnp.where(i < 4, i, 11 - i)


def _fused_ar_relu_quant(partial):
    def body(in_ref, y_ref,
             acc_a, acc_b, stage_a, stage_b, recv_a, recv_b, sxs, sxr,
             ss_a, rs_a, ld_a, st_a, al_a, os_a,
             ss_b, rs_b, ld_b, st_b, al_b, os_b,
             sx_ss, sx_rs, cred_a, cred_b):
        my_id = lax.axis_index("i")
        r = _ring_pos(my_id)
        right_id = _ring_pos((r + 1) % N_DEV)
        left_id = _ring_pos((r + N_DEV - 1) % N_DEV)

        barrier = pltpu.get_barrier_semaphore()
        for nbr in (left_id, right_id):
            pl.semaphore_signal(barrier, inc=1, device_id=(nbr,),
                                device_id_type=pl.DeviceIdType.MESH)
        pl.semaphore_wait(barrier, 2)

        rings = [
            dict(dst=right_id, up=left_id, coff=0, sgn=-1, acc=acc_a,
                 stage=stage_a, recv=recv_a, ss=ss_a, rs=rs_a, ld=ld_a,
                 st=st_a, al=al_a, os=os_a, cred=cred_a),
            dict(dst=left_id, up=right_id, coff=NH, sgn=1, acc=acc_b,
                 stage=stage_b, recv=recv_b, ss=ss_b, rs=rs_b, ld=ld_b,
                 st=st_b, al=al_b, os=os_b, cred=cred_b),
        ]

        def ksl(k):
            return pl.ds(k * NQ, NQ)

        def rchunk(g, s):
            if s <= 6:
                return (r + g["sgn"] * (s + 1)) % N_DEV
            return (r + g["sgn"] * (s - 7)) % N_DEV

        def make_send(g, s, k, src):
            return pltpu.make_async_remote_copy(
                src_ref=src, dst_ref=g["recv"].at[s % 2, :, ksl(k)],
                send_sem=g["ss"].at[s % 2, k], recv_sem=g["rs"].at[s % 2, k],
                device_id=(g["dst"],), device_id_type=pl.DeviceIdType.MESH)

        def make_load(g, s, k):
            return pltpu.make_async_copy(
                in_ref.at[pl.ds(rchunk(g, s) * CH, CH),
                          pl.ds(g["coff"] + k * NQ, NQ)],
                g["stage"].at[s % 2, :, ksl(k)], g["ld"].at[s % 2, k])

        def credit(g):
            pl.semaphore_signal(g["cred"], inc=1, device_id=(g["up"],),
                                device_id_type=pl.DeviceIdType.MESH)

        m = jnp.float32(0.0)
        rd = {}
        st = {}
        own = {}

        accld = []
        for gi, g in enumerate(rings):
            cp = pltpu.make_async_copy(
                in_ref.at[pl.ds(r * CH, CH), pl.ds(g["coff"], NH)],
                g["acc"], g["al"])
            cp.start()
            accld.append(cp)
            for k in (0, 1):
                ld = make_load(g, 0, k)
                ld.start()
                rd[(gi, 0, k, "ld")] = ld
        for gi, g in enumerate(rings):
            accld[gi].wait()
            for k in (0, 1):
                d = make_send(g, 0, k, g["acc"].at[:, ksl(k)])
                rd[(gi, 0, k)] = d
                d.start()

        for s in range(7):
            for gi, g in enumerate(rings):
                for k in (0, 1):
                    d = rd[(gi, s, k)]
                    d.wait_send()
                    d.wait_recv()
                    rd[(gi, s, k, "ld")].wait()
                    if s + 1 <= 6:
                        ld = make_load(g, s + 1, k)
                        ld.start()
                        rd[(gi, s + 1, k, "ld")] = ld
                    g["acc"][:, ksl(k)] = (
                        g["recv"][s % 2, :, k * NQ:(k + 1) * NQ]
                        + g["stage"][s % 2, :, k * NQ:(k + 1) * NQ])
                    credit(g)
                    if s + 1 >= 2:
                        pl.semaphore_wait(g["cred"], 1)
                    d2 = make_send(g, s + 1, k, g["acc"].at[:, ksl(k)])
                    rd[(gi, s + 1, k)] = d2
                    d2.start()
                    if s == 6:
                        m = jnp.maximum(
                            m, jnp.max(g["acc"][:, k * NQ:(k + 1) * NQ]))

        for p, pid in enumerate((my_id ^ 1, my_id ^ 3, my_id ^ 4)):
            sxs[...] = jnp.full((8, 128), m, jnp.float32)
            d = pltpu.make_async_remote_copy(
                src_ref=sxs, dst_ref=sxr.at[p],
                send_sem=sx_ss.at[p], recv_sem=sx_rs.at[p],
                device_id=(pid,), device_id_type=pl.DeviceIdType.MESH)
            d.start()
            d.wait()
            m = jnp.maximum(m, sxr[p, 0, 0])
        amax = jnp.maximum(m, 1e-30)
        inv = 448.0 / amax
        scale = amax / 448.0

        def quant(v):
            q8 = (jnp.maximum(v, 0.0) * inv).astype(jnp.float8_e4m3fn)
            return q8.astype(jnp.float32) * scale

        def make_store(g, s, k, src):
            return pltpu.make_async_copy(
                src, y_ref.at[pl.ds(rchunk(g, s) * CH, CH),
                              pl.ds(g["coff"] + k * NQ, NQ)],
                g["st"].at[s % 2, k])

        for s in range(7, 14):
            for gi, g in enumerate(rings):
                for k in (0, 1):
                    d = rd[(gi, s, k)]
                    d.wait_send()
                    if s >= 8:
                        st[(gi, s - 1, k)].wait()
                        if s - 1 <= LAST_CREDIT:
                            credit(g)
                    d.wait_recv()
                    if s + 1 <= 13:
                        pl.semaphore_wait(g["cred"], 1)
                        d2 = make_send(g, s + 1, k,
                                       g["recv"].at[s % 2, :, ksl(k)])
                        rd[(gi, s + 1, k)] = d2
                        d2.start()
                    if s == 7:
                        g["acc"][:, ksl(k)] = quant(
                            g["acc"][:, k * NQ:(k + 1) * NQ])
                        cp = pltpu.make_async_copy(
                            g["acc"].at[:, ksl(k)],
                            y_ref.at[pl.ds(((r - g["sgn"]) % N_DEV) * CH,
                                           CH),
                                     pl.ds(g["coff"] + k * NQ, NQ)],
                            g["os"].at[k])
                        cp.start()
                        own[(gi, k)] = cp
                    g["stage"][s % 2, :, k * NQ:(k + 1) * NQ] = quant(
                        g["recv"][s % 2, :, k * NQ:(k + 1) * NQ])
                    cp = make_store(g, s, k,
                                    g["stage"].at[s % 2, :, ksl(k)])
                    cp.start()
                    st[(gi, s, k)] = cp

        for gi, g in enumerate(rings):
            for k in (0, 1):
                own[(gi, k)].wait()
                st[(gi, 13, k)].wait()

    return pl.pallas_call(
        body,
        out_shape=jax.ShapeDtypeStruct((M, NOUT), jnp.float32),
        in_specs=[pl.BlockSpec(memory_space=pl.ANY)],
        out_specs=pl.BlockSpec(memory_space=pl.ANY),
        scratch_shapes=[
            pltpu.VMEM((CH, NH), jnp.float32),
            pltpu.VMEM((CH, NH), jnp.float32),
            pltpu.VMEM((2, CH, NH), jnp.float32),
            pltpu.VMEM((2, CH, NH), jnp.float32),
            pltpu.VMEM((2, CH, NH), jnp.float32),
            pltpu.VMEM((2, CH, NH), jnp.float32),
            pltpu.VMEM((8, 128), jnp.float32),
            pltpu.VMEM((3, 8, 128), jnp.float32),
            pltpu.SemaphoreType.DMA((2, 2)),
            pltpu.SemaphoreType.DMA((2, 2)),
            pltpu.SemaphoreType.DMA((2, 2)),
            pltpu.SemaphoreType.DMA((2, 2)),
            pltpu.SemaphoreType.DMA,
            pltpu.SemaphoreType.DMA((2,)),
            pltpu.SemaphoreType.DMA((2, 2)),
            pltpu.SemaphoreType.DMA((2, 2)),
            pltpu.SemaphoreType.DMA((2, 2)),
            pltpu.SemaphoreType.DMA((2, 2)),
            pltpu.SemaphoreType.DMA,
            pltpu.SemaphoreType.DMA((2,)),
            pltpu.SemaphoreType.DMA((3,)),
            pltpu.SemaphoreType.DMA((3,)),
            pltpu.SemaphoreType.REGULAR,
            pltpu.SemaphoreType.REGULAR,
        ],
        compiler_params=pltpu.CompilerParams(collective_id=0),
    )(partial)


def kernel(x, w_mat):
    partial = jnp.dot(x, w_mat, preferred_element_type=jnp.float32,
                      precision=lax.Precision.HIGH)
    return _fused_ar_relu_quant(partial)


# device time: 389587 ns/iter; 1.0085x vs baseline; 1.0085x over previous
import jax
import jax.numpy as jnp
from jax import lax
from jax.experimental import pallas as pl
from jax.experimental.pallas import tpu as pltpu

N_DEV = 8
M, NOUT = 4096, 2048
CH = M // N_DEV
NH = NOUT // 2
NQ = NH // 2
LAST_CREDIT = 11


def _ring_pos(i):
    return jnp.where(i < 4, i, 11 - i)


def _fused_ar_relu_quant(partial):
    def body(in_ref, y_ref,
             acc_a, acc_b, stage_a, stage_b, recv_a, recv_b, sxs, sxr,
             ss_a, rs_a, ld_a, st_a, al_a, os_a,
             ss_b, rs_b, ld_b, st_b, al_b, os_b,
             sx_ss, sx_rs, cred_a, cred_b):
        my_id = lax.axis_index("i")
        r = _ring_pos(my_id)
        right_id = _ring_pos((r + 1) % N_DEV)
        left_id = _ring_pos((r + N_DEV - 1) % N_DEV)

        barrier = pltpu.get_barrier_semaphore()
        for nbr in (left_id, right_id):
            pl.semaphore_signal(barrier, inc=1, device_id=(nbr,),
                                device_id_type=pl.DeviceIdType.MESH)
        pl.semaphore_wait(barrier, 2)

        rings = [
            dict(dst=right_id, up=left_id, coff=0, sgn=-1, acc=acc_a,
                 stage=stage_a, recv=recv_a, ss=ss_a, rs=rs_a, ld=ld_a,
                 st=st_a, al=al_a, os=os_a, cred=cred_a),
            dict(dst=left_id, up=right_id, coff=NH, sgn=1, acc=acc_b,
                 stage=stage_b, recv=recv_b, ss=ss_b, rs=rs_b, ld=ld_b,
                 st=st_b, al=al_b, os=os_b, cred=cred_b),
        ]

        def ksl(k):
            return pl.ds(k * NQ, NQ)

        def rchunk(g, s):
            if s <= 6:
                return (r + g["sgn"] * (s + 1)) % N_DEV
            return (r + g["sgn"] * (s - 7)) % N_DEV

        def make_send(g, s, k, src):
            return pltpu.make_async_remote_copy(
                src_ref=src, dst_ref=g["recv"].at[s % 2, :, ksl(k)],
                send_sem=g["ss"].at[s % 2, k], recv_sem=g["rs"].at[s % 2, k],
                device_id=(g["dst"],), device_id_type=pl.DeviceIdType.MESH)

        def make_load(g, s, k):
            return pltpu.make_async_copy(
                in_ref.at[pl.ds(rchunk(g, s) * CH, CH),
                          pl.ds(g["coff"] + k * NQ, NQ)],
                g["stage"].at[s % 2, :, ksl(k)], g["ld"].at[s % 2, k])

        def credit(g):
            pl.semaphore_signal(g["cred"], inc=1, device_id=(g["up"],),
                                device_id_type=pl.DeviceIdType.MESH)

        m = jnp.float32(0.0)
        rd = {}
        st = {}
        own = {}

        accld = []
        for gi, g in enumerate(rings):
            cp = pltpu.make_async_copy(
                in_ref.at[pl.ds(r * CH, CH), pl.ds(g["coff"], NH)],
                g["acc"], g["al"])
            cp.start()
            accld.append(cp)
            for k in (0, 1):
                ld = make_load(g, 0, k)
                ld.start()
                rd[(gi, 0, k, "ld")] = ld
        for gi, g in enumerate(rings):
            accld[gi].wait()
            for k in (0, 1):
                d = make_send(g, 0, k, g["acc"].at[:, ksl(k)])
                rd[(gi, 0, k)] = d
                d.start()

        for s in range(7):
            for k in (0, 1):
                for gi, g in enumerate(rings):
                    d = rd[(gi, s, k)]
                    d.wait_send()
                    d.wait_recv()
                    rd[(gi, s, k, "ld")].wait()
                    if s + 1 <= 6:
                        ld = make_load(g, s + 1, k)
                        ld.start()
                        rd[(gi, s + 1, k, "ld")] = ld
                    g["acc"][:, ksl(k)] = (
                        g["recv"][s % 2, :, k * NQ:(k + 1) * NQ]
                        + g["stage"][s % 2, :, k * NQ:(k + 1) * NQ])
                    credit(g)
                    if s + 1 >= 2:
                        pl.semaphore_wait(g["cred"], 1)
                    d2 = make_send(g, s + 1, k, g["acc"].at[:, ksl(k)])
                    rd[(gi, s + 1, k)] = d2
                    d2.start()
                    if s == 6:
                        m = jnp.maximum(
                            m, jnp.max(g["acc"][:, k * NQ:(k + 1) * NQ]))

        for p, pid in enumerate((my_id ^ 1, my_id ^ 3, my_id ^ 4)):
            sxs[...] = jnp.full((8, 128), m, jnp.float32)
            d = pltpu.make_async_remote_copy(
                src_ref=sxs, dst_ref=sxr.at[p],
                send_sem=sx_ss.at[p], recv_sem=sx_rs.at[p],
                device_id=(pid,), device_id_type=pl.DeviceIdType.MESH)
            d.start()
            d.wait()
            m = jnp.maximum(m, sxr[p, 0, 0])
        amax = jnp.maximum(m, 1e-30)
        inv = 448.0 / amax
        scale = amax / 448.0

        def quant(v):
            q8 = (jnp.maximum(v, 0.0) * inv).astype(jnp.float8_e4m3fn)
            return q8.astype(jnp.float32) * scale

        def make_store(g, s, k, src):
            return pltpu.make_async_copy(
                src, y_ref.at[pl.ds(rchunk(g, s) * CH, CH),
                              pl.ds(g["coff"] + k * NQ, NQ)],
                g["st"].at[s % 2, k])

        for s in range(7, 14):
            for k in (0, 1):
                for gi, g in enumerate(rings):
                    d = rd[(gi, s, k)]
                    d.wait_send()
                    if s >= 8:
                        st[(gi, s - 1, k)].wait()
                        if s - 1 <= LAST_CREDIT:
                            credit(g)
                    d.wait_recv()
                    if s + 1 <= 13:
                        pl.semaphore_wait(g["cred"], 1)
                        d2 = make_send(g, s + 1, k,
                                       g["recv"].at[s % 2, :, ksl(k)])
                        rd[(gi, s + 1, k)] = d2
                        d2.start()
            for gi, g in enumerate(rings):
                for k in (0, 1):
                    if s == 7:
                        g["acc"][:, ksl(k)] = quant(
                            g["acc"][:, k * NQ:(k + 1) * NQ])
                        cp = pltpu.make_async_copy(
                            g["acc"].at[:, ksl(k)],
                            y_ref.at[pl.ds(((r - g["sgn"]) % N_DEV) * CH,
                                           CH),
                                     pl.ds(g["coff"] + k * NQ, NQ)],
                            g["os"].at[k])
                        cp.start()
                        own[(gi, k)] = cp
                    g["stage"][s % 2, :, k * NQ:(k + 1) * NQ] = quant(
                        g["recv"][s % 2, :, k * NQ:(k + 1) * NQ])
                    cp = make_store(g, s, k,
                                    g["stage"].at[s % 2, :, ksl(k)])
                    cp.start()
                    st[(gi, s, k)] = cp

        for gi, g in enumerate(rings):
            for k in (0, 1):
                own[(gi, k)].wait()
                st[(gi, 13, k)].wait()

    return pl.pallas_call(
        body,
        out_shape=jax.ShapeDtypeStruct((M, NOUT), jnp.float32),
        in_specs=[pl.BlockSpec(memory_space=pl.ANY)],
        out_specs=pl.BlockSpec(memory_space=pl.ANY),
        scratch_shapes=[
            pltpu.VMEM((CH, NH), jnp.float32),
            pltpu.VMEM((CH, NH), jnp.float32),
            pltpu.VMEM((2, CH, NH), jnp.float32),
            pltpu.VMEM((2, CH, NH), jnp.float32),
            pltpu.VMEM((2, CH, NH), jnp.float32),
            pltpu.VMEM((2, CH, NH), jnp.float32),
            pltpu.VMEM((8, 128), jnp.float32),
            pltpu.VMEM((3, 8, 128), jnp.float32),
            pltpu.SemaphoreType.DMA((2, 2)),
            pltpu.SemaphoreType.DMA((2, 2)),
            pltpu.SemaphoreType.DMA((2, 2)),
            pltpu.SemaphoreType.DMA((2, 2)),
            pltpu.SemaphoreType.DMA,
            pltpu.SemaphoreType.DMA((2,)),
            pltpu.SemaphoreType.DMA((2, 2)),
            pltpu.SemaphoreType.DMA((2, 2)),
            pltpu.SemaphoreType.DMA((2, 2)),
            pltpu.SemaphoreType.DMA((2, 2)),
            pltpu.SemaphoreType.DMA,
            pltpu.SemaphoreType.DMA((2,)),
            pltpu.SemaphoreType.DMA((3,)),
            pltpu.SemaphoreType.DMA((3,)),
            pltpu.SemaphoreType.REGULAR,
            pltpu.SemaphoreType.REGULAR,
        ],
        compiler_params=pltpu.CompilerParams(collective_id=0),
    )(partial)


def kernel(x, w_mat):
    partial = jnp.dot(x, w_mat, preferred_element_type=jnp.float32,
                      precision=lax.Precision.HIGH)
    return _fused_ar_relu_quant(partial)


# device time: 364146 ns/iter; 1.0790x vs baseline; 1.0699x over previous
import jax
import jax.numpy as jnp
from jax import lax
from jax.experimental import pallas as pl
from jax.experimental.pallas import tpu as pltpu

N_DEV = 8
M, NOUT = 4096, 2048
K = 512
CH = M // N_DEV
NH = NOUT // 2
NQ = NH // 2
LAST_CREDIT = 11


def _ring_pos(i):
    return jnp.where(i < 4, i, 11 - i)


def _fused_kernel(x, w_mat):
    def body(x_ref, w_ref, y_ref,
             acc_a, acc_b, stage_a, stage_b, recv_a, recv_b, sxs, sxr,
             ss_a, rs_a, st_a, os_a,
             ss_b, rs_b, st_b, os_b,
             sx_ss, sx_rs, cred_a, cred_b):
        my_id = lax.axis_index("i")
        r = _ring_pos(my_id)
        right_id = _ring_pos((r + 1) % N_DEV)
        left_id = _ring_pos((r + N_DEV - 1) % N_DEV)

        barrier = pltpu.get_barrier_semaphore()
        for nbr in (left_id, right_id):
            pl.semaphore_signal(barrier, inc=1, device_id=(nbr,),
                                device_id_type=pl.DeviceIdType.MESH)
        pl.semaphore_wait(barrier, 2)

        rings = [
            dict(dst=right_id, up=left_id, coff=0, sgn=-1, acc=acc_a,
                 stage=stage_a, recv=recv_a, ss=ss_a, rs=rs_a,
                 st=st_a, os=os_a, cred=cred_a),
            dict(dst=left_id, up=right_id, coff=NH, sgn=1, acc=acc_b,
                 stage=stage_b, recv=recv_b, ss=ss_b, rs=rs_b,
                 st=st_b, os=os_b, cred=cred_b),
        ]

        def ksl(k):
            return pl.ds(k * NQ, NQ)

        def rchunk(g, s):
            if s <= 6:
                return (r + g["sgn"] * (s + 1)) % N_DEV
            return (r + g["sgn"] * (s - 7)) % N_DEV

        def pchunk(g, c):
            return jnp.dot(
                x_ref[pl.ds(c * CH, CH), :],
                w_ref[:, g["coff"]:g["coff"] + NH],
                preferred_element_type=jnp.float32,
                precision=lax.Precision.HIGHEST)

        def make_send(g, s, k, src):
            return pltpu.make_async_remote_copy(
                src_ref=src, dst_ref=g["recv"].at[s % 2, :, ksl(k)],
                send_sem=g["ss"].at[s % 2, k], recv_sem=g["rs"].at[s % 2, k],
                device_id=(g["dst"],), device_id_type=pl.DeviceIdType.MESH)

        def credit(g):
            pl.semaphore_signal(g["cred"], inc=1, device_id=(g["up"],),
                                device_id_type=pl.DeviceIdType.MESH)

        m = jnp.float32(0.0)
        rd = {}
        st = {}
        own = {}

        for gi, g in enumerate(rings):
            g["acc"][...] = pchunk(g, r)
            for k in (0, 1):
                d = make_send(g, 0, k, g["acc"].at[:, ksl(k)])
                rd[(gi, 0, k)] = d
                d.start()
        for g in rings:
            g["stage"][0] = pchunk(g, rchunk(g, 0))

        for s in range(7):
            for k in (0, 1):
                for gi, g in enumerate(rings):
                    d = rd[(gi, s, k)]
                    d.wait_send()
                    d.wait_recv()
                    g["acc"][:, ksl(k)] = (
                        g["recv"][s % 2, :, k * NQ:(k + 1) * NQ]
                        + g["stage"][s % 2, :, k * NQ:(k + 1) * NQ])
                    credit(g)
                    if s + 1 >= 2:
                        pl.semaphore_wait(g["cred"], 1)
                    d2 = make_send(g, s + 1, k, g["acc"].at[:, ksl(k)])
                    rd[(gi, s + 1, k)] = d2
                    d2.start()
                    if s == 6:
                        m = jnp.maximum(
                            m, jnp.max(g["acc"][:, k * NQ:(k + 1) * NQ]))
            if s + 1 <= 6:
                for g in rings:
                    g["stage"][(s + 1) % 2] = pchunk(g, rchunk(g, s + 1))

        for p, pid in enumerate((my_id ^ 1, my_id ^ 3, my_id ^ 4)):
            sxs[...] = jnp.full((8, 128), m, jnp.float32)
            d = pltpu.make_async_remote_copy(
                src_ref=sxs, dst_ref=sxr.at[p],
                send_sem=sx_ss.at[p], recv_sem=sx_rs.at[p],
                device_id=(pid,), device_id_type=pl.DeviceIdType.MESH)
            d.start()
            d.wait()
            m = jnp.maximum(m, sxr[p, 0, 0])
        amax = jnp.maximum(m, 1e-30)
        inv = 448.0 / amax
        scale = amax / 448.0

        def quant(v):
            q8 = (jnp.maximum(v, 0.0) * inv).astype(jnp.float8_e4m3fn)
            return q8.astype(jnp.float32) * scale

        def make_store(g, s, k, src):
            return pltpu.make_async_copy(
                src, y_ref.at[pl.ds(rchunk(g, s) * CH, CH),
                              pl.ds(g["coff"] + k * NQ, NQ)],
                g["st"].at[s % 2, k])

        for s in range(7, 14):
            for k in (0, 1):
                for gi, g in enumerate(rings):
                    d = rd[(gi, s, k)]
                    d.wait_send()
                    if s >= 8:
                        st[(gi, s - 1, k)].wait()
                        if s - 1 <= LAST_CREDIT:
                            credit(g)
                    d.wait_recv()
                    if s + 1 <= 13:
                        pl.semaphore_wait(g["cred"], 1)
                        d2 = make_send(g, s + 1, k,
                                       g["recv"].at[s % 2, :, ksl(k)])
                        rd[(gi, s + 1, k)] = d2
                        d2.start()
            for gi, g in enumerate(rings):
                for k in (0, 1):
                    if s == 7:
                        g["acc"][:, ksl(k)] = quant(
                            g["acc"][:, k * NQ:(k + 1) * NQ])
                        cp = pltpu.make_async_copy(
                            g["acc"].at[:, ksl(k)],
                            y_ref.at[pl.ds(((r - g["sgn"]) % N_DEV) * CH,
                                           CH),
                                     pl.ds(g["coff"] + k * NQ, NQ)],
                            g["os"].at[k])
                        cp.start()
                        own[(gi, k)] = cp
                    g["stage"][s % 2, :, k * NQ:(k + 1) * NQ] = quant(
                        g["recv"][s % 2, :, k * NQ:(k + 1) * NQ])
                    cp = make_store(g, s, k,
                                    g["stage"].at[s % 2, :, ksl(k)])
                    cp.start()
                    st[(gi, s, k)] = cp

        for gi, g in enumerate(rings):
            for k in (0, 1):
                own[(gi, k)].wait()
                st[(gi, 13, k)].wait()

    return pl.pallas_call(
        body,
        out_shape=jax.ShapeDtypeStruct((M, NOUT), jnp.float32),
        in_specs=[pl.BlockSpec(memory_space=pltpu.MemorySpace.VMEM),
                  pl.BlockSpec(memory_space=pltpu.MemorySpace.VMEM)],
        out_specs=pl.BlockSpec(memory_space=pl.ANY),
        scratch_shapes=[
            pltpu.VMEM((CH, NH), jnp.float32),
            pltpu.VMEM((CH, NH), jnp.float32),
            pltpu.VMEM((2, CH, NH), jnp.float32),
            pltpu.VMEM((2, CH, NH), jnp.float32),
            pltpu.VMEM((2, CH, NH), jnp.float32),
            pltpu.VMEM((2, CH, NH), jnp.float32),
            pltpu.VMEM((8, 128), jnp.float32),
            pltpu.VMEM((3, 8, 128), jnp.float32),
            pltpu.SemaphoreType.DMA((2, 2)),
            pltpu.SemaphoreType.DMA((2, 2)),
            pltpu.SemaphoreType.DMA((2, 2)),
            pltpu.SemaphoreType.DMA((2,)),
            pltpu.SemaphoreType.DMA((2, 2)),
            pltpu.SemaphoreType.DMA((2, 2)),
            pltpu.SemaphoreType.DMA((2, 2)),
            pltpu.SemaphoreType.DMA((2,)),
            pltpu.SemaphoreType.DMA((3,)),
            pltpu.SemaphoreType.DMA((3,)),
            pltpu.SemaphoreType.REGULAR,
            pltpu.SemaphoreType.REGULAR,
        ],
        compiler_params=pltpu.CompilerParams(collective_id=0),
    )(x, w_mat)


def kernel(x, w_mat):
    return _fused_kernel(x, w_mat)
